# baseline (speedup 1.0000x reference)
import functools

import jax
import jax.numpy as jnp
from jax import lax
from jax.experimental import pallas as pl
from jax.experimental.pallas import tpu as pltpu
from jax.experimental.pallas import tpu_sc as plsc

N = 10000
E = 320000
D = 128
DO = 64

NC = 2
NS = 16
NW = NC * NS
PN = 10240
PE = 327680
EPT = PE // NW
CH = 128
NCHUNK = EPT // CH
RPT = PN // NS

_MESH = plsc.VectorSubcoreMesh(core_axis_name="c", subcore_axis_name="s")
_SC_PARAMS = pltpu.CompilerParams(needs_layout_passes=False)


@functools.partial(
    pl.kernel,
    out_type=jax.ShapeDtypeStruct((NW * PN,), jnp.float32),
    mesh=_MESH,
    compiler_params=_SC_PARAMS,
    scratch_types=[
        pltpu.VMEM((EPT,), jnp.int32),
        pltpu.VMEM((PN,), jnp.float32),
    ],
)
def k_deg(dst_hbm, out_hbm, idx_v, deg_v):
    w = lax.axis_index("s") * NC + lax.axis_index("c")
    pltpu.sync_copy(dst_hbm.at[pl.ds(w * EPT, EPT)], idx_v)
    zero = jnp.zeros((16,), jnp.float32)

    def zbody(j, carry):
        deg_v[pl.ds(j * 16, 16)] = zero
        return carry

    lax.fori_loop(0, PN // 16, zbody, 0)
    ones = jnp.ones((16,), jnp.float32)

    def body(j, carry):
        idx = idx_v[pl.ds(j * 16, 16)]
        plsc.addupdate_scatter(deg_v, [idx], ones)
        return carry

    lax.fori_loop(0, EPT // 16, body, 0)
    pltpu.sync_copy(deg_v, out_hbm.at[pl.ds(w * PN, PN)])


@functools.partial(
    pl.kernel,
    out_type=jax.ShapeDtypeStruct((NC, PN, D), jnp.float32),
    mesh=_MESH,
    compiler_params=_SC_PARAMS,
    scratch_types=[
        pltpu.VMEM((EPT,), jnp.int32),
        pltpu.VMEM((8, CH), jnp.int32),
        pltpu.VMEM((2, CH, D), jnp.float32),
        pltpu.MemorySpace.VMEM_SHARED((PN, D), jnp.float32),
        pltpu.SemaphoreType.DMA,
        pltpu.SemaphoreType.DMA,
        pltpu.SemaphoreType.DMA,
        pltpu.SemaphoreType.DMA,
    ],
)
def k_prop(g_hbm, src_hbm, dst_hbm, out_hbm, src_v, dst_r, buf_v,
           acc_s, sem0, sem1, ssem0, ssem1):
    c = lax.axis_index("c")
    s = lax.axis_index("s")
    w = s * NC + c
    pltpu.sync_copy(src_hbm.at[pl.ds(w * EPT, EPT)], src_v)
    zero = jnp.zeros((16,), jnp.float32)
    for r in range(CH):
        for cc in range(D // 16):
            buf_v[0, r, pl.ds(cc * 16, 16)] = zero
    for k in range(RPT // CH):
        pltpu.sync_copy(buf_v.at[0], acc_s.at[pl.ds(s * RPT + k * CH, CH), :])
    plsc.subcore_barrier()

    gsems = (sem0, sem1)
    ssems = (ssem0, ssem1)
    pend_g = [None, None]
    pend_s = [None, None]
    pend_g[0] = pltpu.async_copy(
        g_hbm.at[src_v.at[pl.ds(0, CH)]], buf_v.at[0], gsems[0])
    pltpu.sync_copy(dst_hbm.at[w, pl.ds(0, 8), :], dst_r)
    for j in range(NCHUNK):
        cb = j % 2
        if j % 8 == 0 and j > 0:
            if pend_s[cb] is not None:
                pend_s[cb].wait()
                pend_s[cb] = None
            if pend_s[1 - cb] is not None:
                pend_s[1 - cb].wait()
                pend_s[1 - cb] = None
            pltpu.sync_copy(dst_hbm.at[w, pl.ds(j, 8), :], dst_r)
        if j + 1 < NCHUNK:
            if pend_s[1 - cb] is not None:
                pend_s[1 - cb].wait()
                pend_s[1 - cb] = None
            pend_g[1 - cb] = pltpu.async_copy(
                g_hbm.at[src_v.at[pl.ds((j + 1) * CH, CH)]],
                buf_v.at[1 - cb], gsems[1 - cb])
        pend_g[cb].wait()
        pend_s[cb] = pltpu.async_copy(
            buf_v.at[cb], acc_s.at[dst_r.at[j % 8]], ssems[cb], add=True)
    for b in range(2):
        if pend_s[b] is not None:
            pend_s[b].wait()

    plsc.subcore_barrier()
    pltpu.sync_copy(acc_s.at[pl.ds(s * RPT, RPT), :],
                    out_hbm.at[c, pl.ds(s * RPT, RPT), :])


_BLK = 1280


def _dinv_of(degp_blk):
    deg = jnp.sum(degp_blk, axis=0) + 1.0
    return lax.rsqrt(deg)[:, None]


def _scale_body(degp_ref, x_ref, o_ref):
    o_ref[...] = x_ref[...] * _dinv_of(degp_ref[...])


def _k_scale(degp, xp):
    return pl.pallas_call(
        _scale_body,
        grid=(PN // _BLK,),
        in_specs=[
            pl.BlockSpec((NW, _BLK), lambda i: (0, i)),
            pl.BlockSpec((_BLK, D), lambda i: (i, 0)),
        ],
        out_specs=pl.BlockSpec((_BLK, D), lambda i: (i, 0)),
        out_shape=jax.ShapeDtypeStruct((PN, D), jnp.float32),
    )(degp, xp)


def _combine_body(degp_ref, s0_ref, s1_ref, g_ref, o_ref):
    dinv = _dinv_of(degp_ref[...])
    o_ref[...] = (s0_ref[...] + s1_ref[...] + g_ref[...]) * (dinv * dinv)


def _k_combine(degp, s0, s1, g):
    return pl.pallas_call(
        _combine_body,
        grid=(PN // _BLK,),
        in_specs=[
            pl.BlockSpec((NW, _BLK), lambda i: (0, i)),
            pl.BlockSpec((_BLK, D), lambda i: (i, 0)),
            pl.BlockSpec((_BLK, D), lambda i: (i, 0)),
            pl.BlockSpec((_BLK, D), lambda i: (i, 0)),
        ],
        out_specs=pl.BlockSpec((_BLK, D), lambda i: (i, 0)),
        out_shape=jax.ShapeDtypeStruct((PN, D), jnp.float32),
    )(degp, s0, s1, g)


_MBLK = 1280


def _mlp_body(degp_ref, s0_ref, s1_ref, g_ref, w1_ref, b1_ref, w2_ref,
              b2_ref, o_ref):
    dinv = _dinv_of(degp_ref[...])
    h = (s0_ref[...] + s1_ref[...] + g_ref[...]) * dinv
    a = lax.dot_general(h, w1_ref[...], (((1,), (1,)), ((), ())),
                        preferred_element_type=jnp.float32)
    a = jnp.maximum(a + b1_ref[...], 0.0)
    z = lax.dot_general(a, w2_ref[...], (((1,), (1,)), ((), ())),
                        preferred_element_type=jnp.float32)
    z = z + b2_ref[...]
    m = jnp.max(z, axis=1, keepdims=True)
    lse = jnp.log(jnp.sum(jnp.exp(z - m), axis=1, keepdims=True)) + m
    o_ref[...] = z - lse


def _k_mlp(degp, s0, s1, g1, W1, b1, W2, b2):
    return pl.pallas_call(
        _mlp_body,
        grid=(PN // _MBLK,),
        in_specs=[
            pl.BlockSpec((NW, _MBLK), lambda i: (0, i)),
            pl.BlockSpec((_MBLK, D), lambda i: (i, 0)),
            pl.BlockSpec((_MBLK, D), lambda i: (i, 0)),
            pl.BlockSpec((_MBLK, D), lambda i: (i, 0)),
            pl.BlockSpec((D, D), lambda i: (0, 0)),
            pl.BlockSpec((1, D), lambda i: (0, 0)),
            pl.BlockSpec((DO, D), lambda i: (0, 0)),
            pl.BlockSpec((1, DO), lambda i: (0, 0)),
        ],
        out_specs=pl.BlockSpec((_MBLK, DO), lambda i: (i, 0)),
        out_shape=jax.ShapeDtypeStruct((PN, DO), jnp.float32),
    )(degp, s0, s1, g1, W1, b1, W2, b2)


def kernel(x, edge_index, W1, b1, W2, b2):
    src = edge_index[0].astype(jnp.int32)
    dst = edge_index[1].astype(jnp.int32)
    padv = jnp.full((PE - E,), PN - 1, jnp.int32)
    src_p = jnp.concatenate([src, padv])
    dst_p = jnp.concatenate([dst, padv])
    xp = jnp.zeros((PN, D), jnp.float32).at[:N].set(x)

    degp = k_deg(dst_p).reshape(NW, PN)
    g0 = _k_scale(degp, xp)
    dst3 = dst_p.reshape(NW, NCHUNK, CH)
    sp1 = k_prop(g0, src_p, dst3)
    g1 = _k_combine(degp, sp1[0], sp1[1], g0)
    sp2 = k_prop(g1, src_p, dst3)
    out = _k_mlp(degp, sp2[0], sp2[1], g1, W1, b1.reshape(1, D),
                 W2, b2.reshape(1, DO))
    return out[:N]


# speedup vs baseline: 2.0574x; 2.0574x over previous
import functools

import jax
import jax.numpy as jnp
from jax import lax
from jax.experimental import pallas as pl
from jax.experimental.pallas import tpu as pltpu
from jax.experimental.pallas import tpu_sc as plsc

N = 10000
E = 320000
D = 128
DH = 64
DO = 64

NC = 2
NS = 16
NW = NC * NS
PN = 10240
PE = 327680
EPT = PE // NW
CH = 128
NCHUNK = EPT // CH
RPT = PN // NS

_MESH = plsc.VectorSubcoreMesh(core_axis_name="c", subcore_axis_name="s")
_SC_PARAMS = pltpu.CompilerParams(needs_layout_passes=False,
                                  use_tc_tiling_on_sc=False)


@functools.partial(
    pl.kernel,
    out_type=jax.ShapeDtypeStruct((NW * PN,), jnp.float32),
    mesh=_MESH,
    compiler_params=_SC_PARAMS,
    scratch_types=[
        pltpu.VMEM((EPT,), jnp.int32),
        pltpu.VMEM((PN,), jnp.float32),
    ],
)
def k_deg(dst_hbm, out_hbm, idx_v, deg_v):
    w = lax.axis_index("s") * NC + lax.axis_index("c")
    pltpu.sync_copy(dst_hbm.at[pl.ds(w * EPT, EPT)], idx_v)
    zero = jnp.zeros((16,), jnp.float32)

    def zbody(j, carry):
        deg_v[pl.ds(j * 16, 16)] = zero
        return carry

    lax.fori_loop(0, PN // 16, zbody, 0)
    ones = jnp.ones((16,), jnp.float32)

    def body(j, carry):
        idx = idx_v[pl.ds(j * 16, 16)]
        plsc.addupdate_scatter(deg_v, [idx], ones)
        return carry

    lax.fori_loop(0, EPT // 16, body, 0)
    pltpu.sync_copy(deg_v, out_hbm.at[pl.ds(w * PN, PN)])


@functools.partial(
    pl.kernel,
    out_type=jax.ShapeDtypeStruct((NC, 2, PN, DH), jnp.float32),
    mesh=_MESH,
    compiler_params=_SC_PARAMS,
    scratch_types=[
        pltpu.VMEM((EPT,), jnp.int32),
        pltpu.VMEM((8, CH), jnp.int32),
        pltpu.VMEM((2, CH, DH), jnp.float32),
        pltpu.MemorySpace.VMEM_SHARED((PN, DH), jnp.float32),
        pltpu.MemorySpace.VMEM_SHARED((PN, DH), jnp.float32),
        pltpu.SemaphoreType.DMA,
        pltpu.SemaphoreType.DMA,
        pltpu.SemaphoreType.DMA,
        pltpu.SemaphoreType.DMA,
    ],
)
def k_prop(g_lo_hbm, g_hi_hbm, src_hbm, dst_hbm, out_hbm, src_v, dst_r, buf_v,
           g_sp, acc_s, sem0, sem1, ssem0, ssem1):
    c = lax.axis_index("c")
    s = lax.axis_index("s")
    w = s * NC + c
    rows = pl.ds(s * RPT, RPT)
    pltpu.sync_copy(src_hbm.at[pl.ds(w * EPT, EPT)], src_v)
    gsems = (sem0, sem1)
    ssems = (ssem0, ssem1)

    for half, g_in in ((0, g_lo_hbm), (1, g_hi_hbm)):
        for k in range(RPT // CH):
            rk = pl.ds(s * RPT + k * CH, CH)
            pltpu.sync_copy(g_in.at[rk, :], buf_v.at[1])
            pltpu.sync_copy(buf_v.at[1], g_sp.at[rk, :])
        zero = jnp.zeros((16,), jnp.float32)
        for r in range(CH):
            for cc in range(DH // 16):
                buf_v[0, r, pl.ds(cc * 16, 16)] = zero
        for k in range(RPT // CH):
            pltpu.sync_copy(buf_v.at[0],
                            acc_s.at[pl.ds(s * RPT + k * CH, CH), :])
        plsc.subcore_barrier()

        pend_s = [None, None]
        pend_g = [None, None]
        pend_g[0] = pltpu.async_copy(
            g_sp.at[src_v.at[pl.ds(0, CH)]], buf_v.at[0], gsems[0])
        pltpu.sync_copy(dst_hbm.at[w, pl.ds(0, 8), :], dst_r)
        for j in range(NCHUNK):
            cb = j % 2
            if j % 8 == 0 and j > 0:
                for b in range(2):
                    if pend_s[b] is not None:
                        pend_s[b].wait()
                        pend_s[b] = None
                pltpu.sync_copy(dst_hbm.at[w, pl.ds(j, 8), :], dst_r)
            if j + 1 < NCHUNK:
                if pend_s[1 - cb] is not None:
                    pend_s[1 - cb].wait()
                    pend_s[1 - cb] = None
                pend_g[1 - cb] = pltpu.async_copy(
                    g_sp.at[src_v.at[pl.ds((j + 1) * CH, CH)]],
                    buf_v.at[1 - cb], gsems[1 - cb])
            pend_g[cb].wait()
            pend_s[cb] = pltpu.async_copy(
                buf_v.at[cb], acc_s.at[dst_r.at[j % 8]], ssems[cb], add=True)
        for b in range(2):
            if pend_s[b] is not None:
                pend_s[b].wait()

        plsc.subcore_barrier()
        pltpu.sync_copy(acc_s.at[rows, :], out_hbm.at[c, half, rows, :])


_BLK = 1280
_ROWS = pl.BlockSpec((_BLK, DH), lambda i: (i, 0))
_DEGS = pl.BlockSpec((NW, _BLK), lambda i: (0, i))


def _dinv_of(degp_blk):
    deg = jnp.sum(degp_blk, axis=0) + 1.0
    return lax.rsqrt(deg)[:, None]


def _scale_body(degp_ref, x_ref, lo_ref, hi_ref):
    dinv = _dinv_of(degp_ref[...])
    lo_ref[...] = x_ref[:, :DH] * dinv
    hi_ref[...] = x_ref[:, DH:] * dinv


def _k_scale(degp, xp):
    return pl.pallas_call(
        _scale_body,
        grid=(PN // _BLK,),
        in_specs=[_DEGS, pl.BlockSpec((_BLK, D), lambda i: (i, 0))],
        out_specs=(_ROWS, _ROWS),
        out_shape=(jax.ShapeDtypeStruct((PN, DH), jnp.float32),
                   jax.ShapeDtypeStruct((PN, DH), jnp.float32)),
    )(degp, xp)


def _combine_body(degp_ref, s0lo_ref, s1lo_ref, glo_ref,
                  s0hi_ref, s1hi_ref, ghi_ref, lo_ref, hi_ref):
    dinv = _dinv_of(degp_ref[...])
    d2 = dinv * dinv
    lo_ref[...] = (s0lo_ref[...] + s1lo_ref[...] + glo_ref[...]) * d2
    hi_ref[...] = (s0hi_ref[...] + s1hi_ref[...] + ghi_ref[...]) * d2


def _k_combine(degp, s0lo, s1lo, glo, s0hi, s1hi, ghi):
    return pl.pallas_call(
        _combine_body,
        grid=(PN // _BLK,),
        in_specs=[_DEGS] + [_ROWS] * 6,
        out_specs=(_ROWS, _ROWS),
        out_shape=(jax.ShapeDtypeStruct((PN, DH), jnp.float32),
                   jax.ShapeDtypeStruct((PN, DH), jnp.float32)),
    )(degp, s0lo, s1lo, glo, s0hi, s1hi, ghi)


def _mlp_body(degp_ref, s0lo_ref, s1lo_ref, glo_ref, s0hi_ref, s1hi_ref,
              ghi_ref, w1_ref, b1_ref, w2_ref, b2_ref, o_ref):
    dinv = _dinv_of(degp_ref[...])
    h_lo = (s0lo_ref[...] + s1lo_ref[...] + glo_ref[...]) * dinv
    h_hi = (s0hi_ref[...] + s1hi_ref[...] + ghi_ref[...]) * dinv
    h = jnp.concatenate([h_lo, h_hi], axis=1)
    a = lax.dot_general(h, w1_ref[...], (((1,), (1,)), ((), ())),
                        preferred_element_type=jnp.float32)
    a = jnp.maximum(a + b1_ref[...], 0.0)
    z = lax.dot_general(a, w2_ref[...], (((1,), (1,)), ((), ())),
                        preferred_element_type=jnp.float32)
    z = z + b2_ref[...]
    m = jnp.max(z, axis=1, keepdims=True)
    lse = jnp.log(jnp.sum(jnp.exp(z - m), axis=1, keepdims=True)) + m
    o_ref[...] = z - lse


def _k_mlp(degp, s0lo, s1lo, glo, s0hi, s1hi, ghi, W1, b1, W2, b2):
    return pl.pallas_call(
        _mlp_body,
        grid=(PN // _BLK,),
        in_specs=[_DEGS] + [_ROWS] * 6 + [
            pl.BlockSpec((D, D), lambda i: (0, 0)),
            pl.BlockSpec((1, D), lambda i: (0, 0)),
            pl.BlockSpec((DO, D), lambda i: (0, 0)),
            pl.BlockSpec((1, DO), lambda i: (0, 0)),
        ],
        out_specs=pl.BlockSpec((_BLK, DO), lambda i: (i, 0)),
        out_shape=jax.ShapeDtypeStruct((PN, DO), jnp.float32),
    )(degp, s0lo, s1lo, glo, s0hi, s1hi, ghi, W1, b1, W2, b2)


def kernel(x, edge_index, W1, b1, W2, b2):
    src = edge_index[0].astype(jnp.int32)
    dst = edge_index[1].astype(jnp.int32)
    padv = jnp.full((PE - E,), PN - 1, jnp.int32)
    src_p = jnp.concatenate([src, padv])
    dst_p = jnp.concatenate([dst, padv])
    xp = jnp.zeros((PN, D), jnp.float32).at[:N].set(x)

    degp = k_deg(dst_p).reshape(NW, PN)
    g0lo, g0hi = _k_scale(degp, xp)
    dst3 = dst_p.reshape(NW, NCHUNK, CH)
    sp1 = k_prop(g0lo, g0hi, src_p, dst3)
    g1lo, g1hi = _k_combine(degp, sp1[0, 0], sp1[1, 0], g0lo,
                            sp1[0, 1], sp1[1, 1], g0hi)
    sp2 = k_prop(g1lo, g1hi, src_p, dst3)
    out = _k_mlp(degp, sp2[0, 0], sp2[1, 0], g1lo, sp2[0, 1], sp2[1, 1], g1hi,
                 W1, b1.reshape(1, D), W2, b2.reshape(1, DO))
    return out[:N]


# speedup vs baseline: 2.2367x; 1.0872x over previous
import functools

import jax
import jax.numpy as jnp
from jax import lax
from jax.experimental import pallas as pl
from jax.experimental.pallas import tpu as pltpu
from jax.experimental.pallas import tpu_sc as plsc

N = 10000
E = 320000
D = 128
DH = 64
DO = 64

NC = 2
NS = 16
NW = NC * NS
PN = 10240
PE = 327680
EPT = PE // NW
CH = 128
NCHUNK = EPT // CH
RPT = PN // NS

_MESH = plsc.VectorSubcoreMesh(core_axis_name="c", subcore_axis_name="s")
_SC_PARAMS = pltpu.CompilerParams(needs_layout_passes=False,
                                  use_tc_tiling_on_sc=False)


@functools.partial(
    pl.kernel,
    out_type=jax.ShapeDtypeStruct((NW * PN,), jnp.float32),
    mesh=_MESH,
    compiler_params=_SC_PARAMS,
    scratch_types=[
        pltpu.VMEM((EPT,), jnp.int32),
        pltpu.VMEM((PN,), jnp.float32),
    ],
)
def k_deg(dst_hbm, out_hbm, idx_v, deg_v):
    w = lax.axis_index("s") * NC + lax.axis_index("c")
    pltpu.sync_copy(dst_hbm.at[pl.ds(w * EPT, EPT)], idx_v)
    zero = jnp.zeros((16,), jnp.float32)

    def zbody(j, carry):
        deg_v[pl.ds(j * 16, 16)] = zero
        return carry

    lax.fori_loop(0, PN // 16, zbody, 0)
    ones = jnp.ones((16,), jnp.float32)

    def body(j, carry):
        idx = idx_v[pl.ds(j * 16, 16)]
        plsc.addupdate_scatter(deg_v, [idx], ones)
        return carry

    lax.fori_loop(0, EPT // 16, body, 0)
    pltpu.sync_copy(deg_v, out_hbm.at[pl.ds(w * PN, PN)])


@functools.partial(
    pl.kernel,
    out_type=jax.ShapeDtypeStruct((NC, 2, PN, DH), jnp.float32),
    mesh=_MESH,
    compiler_params=_SC_PARAMS,
    scratch_types=[
        pltpu.VMEM((EPT,), jnp.int32),
        pltpu.VMEM((NCHUNK, CH), jnp.int32),
        pltpu.VMEM((2, CH, DH), jnp.float32),
        pltpu.MemorySpace.VMEM_SHARED((PN, DH), jnp.float32),
        pltpu.MemorySpace.VMEM_SHARED((PN, DH), jnp.float32),
        pltpu.SemaphoreType.DMA,
        pltpu.SemaphoreType.DMA,
        pltpu.SemaphoreType.DMA,
        pltpu.SemaphoreType.DMA,
    ],
)
def k_prop(g_lo_hbm, g_hi_hbm, src_hbm, dst_hbm, out_hbm, src_v, dst_v, buf_v,
           g_sp, acc_s, sem0, sem1, ssem0, ssem1):
    c = lax.axis_index("c")
    s = lax.axis_index("s")
    w = s * NC + c
    rows = pl.ds(s * RPT, RPT)
    gsems = (sem0, sem1)
    ssems = (ssem0, ssem1)
    pltpu.sync_copy(src_hbm.at[pl.ds(w * EPT, EPT)], src_v)
    pltpu.sync_copy(dst_hbm.at[w], dst_v)
    zero = jnp.zeros((16,), jnp.float32)

    for half, g_in in ((0, g_lo_hbm), (1, g_hi_hbm)):
        for r in range(CH):
            for cc in range(DH // 16):
                buf_v[0, r, pl.ds(cc * 16, 16)] = zero
        pend = []
        for k in range(RPT // CH):
            rk = pl.ds(s * RPT + k * CH, CH)
            pend.append(pltpu.async_copy(g_in.at[rk, :], g_sp.at[rk, :],
                                         gsems[0]))
            pend.append(pltpu.async_copy(buf_v.at[0], acc_s.at[rk, :],
                                         ssems[0]))
        for p in pend:
            p.wait()
        plsc.subcore_barrier()

        pend_s = [None, None]
        pend_g = [None, None]
        pend_g[0] = pltpu.async_copy(
            g_sp.at[src_v.at[pl.ds(0, CH)]], buf_v.at[0], gsems[0])
        for j in range(NCHUNK):
            cb = j % 2
            if j + 1 < NCHUNK:
                if pend_s[1 - cb] is not None:
                    pend_s[1 - cb].wait()
                    pend_s[1 - cb] = None
                pend_g[1 - cb] = pltpu.async_copy(
                    g_sp.at[src_v.at[pl.ds((j + 1) * CH, CH)]],
                    buf_v.at[1 - cb], gsems[1 - cb])
            pend_g[cb].wait()
            pend_s[cb] = pltpu.async_copy(
                buf_v.at[cb], acc_s.at[dst_v.at[j]], ssems[cb], add=True)
        for b in range(2):
            if pend_s[b] is not None:
                pend_s[b].wait()

        plsc.subcore_barrier()
        pltpu.sync_copy(acc_s.at[rows, :], out_hbm.at[c, half, rows, :])


_BLK = 1280
_ROWS = pl.BlockSpec((_BLK, DH), lambda i: (i, 0))
_DEGS = pl.BlockSpec((NW, _BLK), lambda i: (0, i))


def _dinv_of(degp_blk):
    deg = jnp.sum(degp_blk, axis=0) + 1.0
    return lax.rsqrt(deg)[:, None]


def _scale_body(degp_ref, x_ref, lo_ref, hi_ref):
    dinv = _dinv_of(degp_ref[...])
    lo_ref[...] = x_ref[:, :DH] * dinv
    hi_ref[...] = x_ref[:, DH:] * dinv


def _k_scale(degp, xp):
    return pl.pallas_call(
        _scale_body,
        grid=(PN // _BLK,),
        in_specs=[_DEGS, pl.BlockSpec((_BLK, D), lambda i: (i, 0))],
        out_specs=(_ROWS, _ROWS),
        out_shape=(jax.ShapeDtypeStruct((PN, DH), jnp.float32),
                   jax.ShapeDtypeStruct((PN, DH), jnp.float32)),
    )(degp, xp)


def _combine_body(degp_ref, s0lo_ref, s1lo_ref, glo_ref,
                  s0hi_ref, s1hi_ref, ghi_ref, lo_ref, hi_ref):
    dinv = _dinv_of(degp_ref[...])
    d2 = dinv * dinv
    lo_ref[...] = (s0lo_ref[...] + s1lo_ref[...] + glo_ref[...]) * d2
    hi_ref[...] = (s0hi_ref[...] + s1hi_ref[...] + ghi_ref[...]) * d2


def _k_combine(degp, s0lo, s1lo, glo, s0hi, s1hi, ghi):
    return pl.pallas_call(
        _combine_body,
        grid=(PN // _BLK,),
        in_specs=[_DEGS] + [_ROWS] * 6,
        out_specs=(_ROWS, _ROWS),
        out_shape=(jax.ShapeDtypeStruct((PN, DH), jnp.float32),
                   jax.ShapeDtypeStruct((PN, DH), jnp.float32)),
    )(degp, s0lo, s1lo, glo, s0hi, s1hi, ghi)


def _mlp_body(degp_ref, s0lo_ref, s1lo_ref, glo_ref, s0hi_ref, s1hi_ref,
              ghi_ref, w1_ref, b1_ref, w2_ref, b2_ref, o_ref):
    dinv = _dinv_of(degp_ref[...])
    h_lo = (s0lo_ref[...] + s1lo_ref[...] + glo_ref[...]) * dinv
    h_hi = (s0hi_ref[...] + s1hi_ref[...] + ghi_ref[...]) * dinv
    h = jnp.concatenate([h_lo, h_hi], axis=1)
    a = lax.dot_general(h, w1_ref[...], (((1,), (1,)), ((), ())),
                        preferred_element_type=jnp.float32)
    a = jnp.maximum(a + b1_ref[...], 0.0)
    z = lax.dot_general(a, w2_ref[...], (((1,), (1,)), ((), ())),
                        preferred_element_type=jnp.float32)
    z = z + b2_ref[...]
    m = jnp.max(z, axis=1, keepdims=True)
    lse = jnp.log(jnp.sum(jnp.exp(z - m), axis=1, keepdims=True)) + m
    o_ref[...] = z - lse


def _k_mlp(degp, s0lo, s1lo, glo, s0hi, s1hi, ghi, W1, b1, W2, b2):
    return pl.pallas_call(
        _mlp_body,
        grid=(PN // _BLK,),
        in_specs=[_DEGS] + [_ROWS] * 6 + [
            pl.BlockSpec((D, D), lambda i: (0, 0)),
            pl.BlockSpec((1, D), lambda i: (0, 0)),
            pl.BlockSpec((DO, D), lambda i: (0, 0)),
            pl.BlockSpec((1, DO), lambda i: (0, 0)),
        ],
        out_specs=pl.BlockSpec((_BLK, DO), lambda i: (i, 0)),
        out_shape=jax.ShapeDtypeStruct((PN, DO), jnp.float32),
    )(degp, s0lo, s1lo, glo, s0hi, s1hi, ghi, W1, b1, W2, b2)


def kernel(x, edge_index, W1, b1, W2, b2):
    src = edge_index[0].astype(jnp.int32)
    dst = edge_index[1].astype(jnp.int32)
    padv = jnp.full((PE - E,), PN - 1, jnp.int32)
    src_p = jnp.concatenate([src, padv])
    dst_p = jnp.concatenate([dst, padv])
    xp = jnp.zeros((PN, D), jnp.float32).at[:N].set(x)

    degp = k_deg(dst_p).reshape(NW, PN)
    g0lo, g0hi = _k_scale(degp, xp)
    dst3 = dst_p.reshape(NW, NCHUNK, CH)
    sp1 = k_prop(g0lo, g0hi, src_p, dst3)
    g1lo, g1hi = _k_combine(degp, sp1[0, 0], sp1[1, 0], g0lo,
                            sp1[0, 1], sp1[1, 1], g0hi)
    sp2 = k_prop(g1lo, g1hi, src_p, dst3)
    out = _k_mlp(degp, sp2[0, 0], sp2[1, 0], g1lo, sp2[0, 1], sp2[1, 1], g1hi,
                 W1, b1.reshape(1, D), W2, b2.reshape(1, DO))
    return out[:N]
